# baseline (device time: 42880 ns/iter reference)
import jax
import jax.numpy as jnp
from jax import lax
from jax.experimental import pallas as pl
from jax.experimental.pallas import tpu as pltpu

N_DEV = 8
N_TOK = 1024
D_IN = 256
D_OUT = 512
E_LOCAL = 4
N_EXP = 32
CHUNK = N_TOK // N_DEV


def kernel(x, router_W, route_idx, expert_W):
    def body(x_ref, rw_ref, idx_ref, ew_ref, out_ref,
             part_ref, comm_ref, send_sems, recv_sems):
        my = lax.axis_index("i")
        left = lax.rem(my + N_DEV - 1, N_DEV)
        right = lax.rem(my + 1, N_DEV)

        barrier_sem = pltpu.get_barrier_semaphore()
        for nbr in (left, right):
            pl.semaphore_signal(
                barrier_sem, inc=1,
                device_id=(nbr,), device_id_type=pl.DeviceIdType.MESH,
            )
        pl.semaphore_wait(barrier_sem, 2)

        xf = x_ref[:, :]
        scores = jnp.dot(xf, rw_ref[:, :], preferred_element_type=jnp.float32)
        m = jnp.max(scores, axis=-1, keepdims=True)
        p = jnp.exp(scores - m)
        eids = lax.broadcasted_iota(jnp.int32, (N_TOK, N_EXP), 1)
        i0 = idx_ref[:, 0:1]
        i1 = idx_ref[:, 1:2]
        p0 = jnp.sum(jnp.where(eids == i0, p, 0.0), axis=1, keepdims=True)
        p1 = jnp.sum(jnp.where(eids == i1, p, 0.0), axis=1, keepdims=True)
        gs = p0 + p1
        w0 = p0 / gs
        w1 = p1 / gs

        part_ref[:, :] = jnp.zeros((N_TOK, D_OUT), jnp.float32)
        for k in range(E_LOCAL):
            e = my * E_LOCAL + k
            ck = (w0 * (i0 == e).astype(jnp.float32)
                  + w1 * (i1 == e).astype(jnp.float32))
            xk = (xf * ck).astype(jnp.bfloat16)
            part_ref[:, :] += jnp.dot(
                xk, ew_ref[k, :, :].astype(jnp.bfloat16),
                preferred_element_type=jnp.float32,
            )

        comm_ref[0, :, :] = part_ref[pl.ds(left * CHUNK, CHUNK), :]
        for s in range(N_DEV - 1):
            rdma = pltpu.make_async_remote_copy(
                src_ref=comm_ref.at[s],
                dst_ref=comm_ref.at[s + 1],
                send_sem=send_sems.at[s],
                recv_sem=recv_sems.at[s + 1],
                device_id=(right,),
                device_id_type=pl.DeviceIdType.MESH,
            )
            rdma.start()
            rdma.wait()
            c = lax.rem(my + N_DEV - 2 - s + N_DEV, N_DEV)
            comm_ref[s + 1, :, :] += part_ref[pl.ds(c * CHUNK, CHUNK), :]
        out_ref[:, :] = comm_ref[N_DEV - 1, :, :]

    return pl.pallas_call(
        body,
        out_shape=jax.ShapeDtypeStruct((CHUNK, D_OUT), jnp.float32),
        in_specs=[pl.BlockSpec(memory_space=pltpu.VMEM)] * 4,
        out_specs=pl.BlockSpec(memory_space=pltpu.VMEM),
        scratch_shapes=[
            pltpu.VMEM((N_TOK, D_OUT), jnp.float32),
            pltpu.VMEM((N_DEV, CHUNK, D_OUT), jnp.float32),
            pltpu.SemaphoreType.DMA((N_DEV,)),
            pltpu.SemaphoreType.DMA((N_DEV,)),
        ],
        compiler_params=pltpu.CompilerParams(collective_id=0),
    )(x, router_W, route_idx, expert_W)


# device time: 20623 ns/iter; 2.0792x vs baseline; 2.0792x over previous
import jax
import jax.numpy as jnp
from jax import lax
from jax.experimental import pallas as pl
from jax.experimental.pallas import tpu as pltpu

N_DEV = 8
N_TOK = 1024
D_IN = 256
D_OUT = 512
E_LOCAL = 4
N_EXP = 32
CHUNK = N_TOK // N_DEV


def kernel(x, router_W, route_idx, expert_W):
    def body(x_ref, rw_ref, idx_ref, ew_ref, out_ref,
             part_ref, send_ref, recv_ref, send_sems, recv_sems):
        my = lax.axis_index("i")

        barrier_sem = pltpu.get_barrier_semaphore()
        for d in range(1, N_DEV):
            pl.semaphore_signal(
                barrier_sem, inc=1,
                device_id=(lax.rem(my + d, N_DEV),),
                device_id_type=pl.DeviceIdType.MESH,
            )
        pl.semaphore_wait(barrier_sem, N_DEV - 1)

        xf = x_ref[:, :]
        scores = jnp.dot(
            xf.astype(jnp.bfloat16), rw_ref[:, :].astype(jnp.bfloat16),
            preferred_element_type=jnp.float32,
        )
        m = jnp.max(scores, axis=-1, keepdims=True)
        p = jnp.exp(scores - m)
        eids = lax.broadcasted_iota(jnp.int32, (N_TOK, N_EXP), 1)
        i0 = idx_ref[:, 0:1]
        i1 = idx_ref[:, 1:2]
        p0 = jnp.sum(jnp.where(eids == i0, p, 0.0), axis=1, keepdims=True)
        p1 = jnp.sum(jnp.where(eids == i1, p, 0.0), axis=1, keepdims=True)
        gs = p0 + p1
        w0 = p0 / gs
        w1 = p1 / gs

        part = jnp.zeros((N_TOK, D_OUT), jnp.float32)
        for k in range(E_LOCAL):
            e = my * E_LOCAL + k
            ck = (w0 * (i0 == e).astype(jnp.float32)
                  + w1 * (i1 == e).astype(jnp.float32))
            xk = (xf * ck).astype(jnp.bfloat16)
            part = part + jnp.dot(
                xk, ew_ref[k, :, :].astype(jnp.bfloat16),
                preferred_element_type=jnp.float32,
            )

        part_ref[:, :] = part
        rdmas = []
        for d in range(1, N_DEV):
            t = lax.rem(my + d, N_DEV)
            chunk = part_ref[pl.ds(t * CHUNK, CHUNK), :]
            send_ref[d, :, :] = chunk.astype(jnp.bfloat16)
            rdma = pltpu.make_async_remote_copy(
                src_ref=send_ref.at[d],
                dst_ref=recv_ref.at[d],
                send_sem=send_sems.at[d],
                recv_sem=recv_sems.at[d],
                device_id=(t,),
                device_id_type=pl.DeviceIdType.MESH,
            )
            rdma.start()
            rdmas.append(rdma)

        acc = part_ref[pl.ds(my * CHUNK, CHUNK), :]
        for d in range(1, N_DEV):
            rdmas[d - 1].wait_recv()
            acc = acc + recv_ref[d, :, :].astype(jnp.float32)
        out_ref[:, :] = acc
        for r in rdmas:
            r.wait_send()

    return pl.pallas_call(
        body,
        out_shape=jax.ShapeDtypeStruct((CHUNK, D_OUT), jnp.float32),
        in_specs=[pl.BlockSpec(memory_space=pltpu.VMEM)] * 4,
        out_specs=pl.BlockSpec(memory_space=pltpu.VMEM),
        scratch_shapes=[
            pltpu.VMEM((N_TOK, D_OUT), jnp.float32),
            pltpu.VMEM((N_DEV, CHUNK, D_OUT), jnp.bfloat16),
            pltpu.VMEM((N_DEV, CHUNK, D_OUT), jnp.bfloat16),
            pltpu.SemaphoreType.DMA((N_DEV,)),
            pltpu.SemaphoreType.DMA((N_DEV,)),
        ],
        compiler_params=pltpu.CompilerParams(collective_id=0),
    )(x, router_W, route_idx, expert_W)
